# Initial kernel scaffold; baseline (speedup 1.0000x reference)
import jax
import jax.numpy as jnp
from jax.experimental import pallas as pl


def kernel(x, edge_index, W1, b1, W2, b2, W3, b3, W4, b4, W5, b5, W6, b6, W7, b7, W8, b8, W9, b9, W10, b10):
    raise NotImplementedError("write your pallas kernel here")


# speedup vs baseline: 1.6915x; 1.6915x over previous
import functools

import jax
import jax.numpy as jnp
from jax.experimental import pallas as pl

N_NODES = 10000
BN = 400


def _mm_body(x_ref, w_ref, b_ref, sin_ref, sout_ref, o_ref, *, relu):
    x = x_ref[...]
    if sin_ref is not None:
        x = x * sin_ref[...]
    acc = jnp.dot(x, w_ref[...], preferred_element_type=jnp.float32)
    acc = acc + b_ref[...]
    if relu:
        acc = jnp.maximum(acc, 0.0)
    if sout_ref is not None:
        acc = acc * sout_ref[...]
    o_ref[...] = acc


def _dense(x, W, b, *, relu, scale_in=None, scale_out=None):
    n, din = x.shape
    dout = W.shape[1]
    b2 = b.reshape(1, dout)
    grid = (n // BN,)
    in_specs = [
        pl.BlockSpec((BN, din), lambda i: (i, 0)),
        pl.BlockSpec((din, dout), lambda i: (0, 0)),
        pl.BlockSpec((1, dout), lambda i: (0, 0)),
    ]
    args = [x, W, b2]
    for s in (scale_in, scale_out):
        if s is None:
            in_specs.append(None)
            args.append(None)
        else:
            in_specs.append(pl.BlockSpec((BN, 1), lambda i: (i, 0)))
            args.append(s.reshape(n, 1))
    have_sin = scale_in is not None
    have_sout = scale_out is not None

    def body(*refs):
        x_ref, w_ref, b_ref = refs[0], refs[1], refs[2]
        k = 3
        sin_ref = refs[k] if have_sin else None
        k += 1 if have_sin else 0
        sout_ref = refs[k] if have_sout else None
        _mm_body(x_ref, w_ref, b_ref, sin_ref, sout_ref, refs[-1], relu=relu)

    return pl.pallas_call(
        body,
        grid=grid,
        in_specs=[s for s in in_specs if s is not None],
        out_specs=pl.BlockSpec((BN, dout), lambda i: (i, 0)),
        out_shape=jax.ShapeDtypeStruct((n, dout), jnp.float32),
    )(*[a for a in args if a is not None])


def _segment_sum(u, src, dst):
    msg = u[src]
    return jnp.zeros_like(u).at[dst].add(msg)


def kernel(x, edge_index, W1, b1, W2, b2, W3, b3, W4, b4, W5, b5,
           W6, b6, W7, b7, W8, b8, W9, b9, W10, b10):
    src = edge_index[0].astype(jnp.int32)
    dst = edge_index[1].astype(jnp.int32)
    ones = jnp.ones((src.shape[0],), dtype=jnp.float32)
    deg = jnp.zeros((N_NODES,), dtype=jnp.float32).at[dst].add(ones) + 1.0
    dis = deg ** -0.5

    Ws = [W1, W2, W3, W4, W5, W6, W7, W8, W9, W10]
    bs = [b1, b2, b3, b4, b5, b6, b7, b8, b9, b10]

    h = x
    for i in range(10):
        W, b = Ws[i], bs[i]
        din, dout = W.shape
        relu = i < 9
        if din <= dout:
            u = h * dis[:, None]
            v = _segment_sum(u, src, dst) + u
            h = _dense(v, W, b, relu=relu, scale_in=dis)
        else:
            u = _dense(h, W, b * 0.0, relu=False, scale_out=dis)
            v = _segment_sum(u, src, dst) + u
            h = v * dis[:, None] + b[None, :]
            if relu:
                h = jnp.maximum(h, 0.0)
    return h


# speedup vs baseline: 4.9127x; 2.9044x over previous
import functools

import jax
import jax.numpy as jnp
from jax import lax
from jax.experimental import pallas as pl
from jax.experimental.pallas import tpu as pltpu
from jax.experimental.pallas import tpu_sc as plsc

N_NODES = 10000
N_EDGES = 320000
NP = 10240
BN = 512
NS = 16
NC = 2
RPT = NP // NS
EC = 100
NCH = N_EDGES // NS // EC


def _dense_split(h2, W, b, *, relu, scale_in=None, pre_bias=None,
                 pre_relu=False, scale_out=None):
    _, n, dhi = h2.shape
    din, dout = W.shape
    dho = dout // 2
    W2 = W.reshape(din, 2, dho).transpose(1, 0, 2)
    b2 = b.reshape(2, 1, dho)
    have_si = scale_in is not None
    have_pb = pre_bias is not None
    have_so = scale_out is not None

    def body(*refs):
        x2_ref, w_ref, b_ref = refs[0], refs[1], refs[2]
        k = 3
        si_ref = pb_ref = so_ref = None
        if have_si:
            si_ref = refs[k]; k += 1
        if have_pb:
            pb_ref = refs[k]; k += 1
        if have_so:
            so_ref = refs[k]; k += 1
        o_ref = refs[-1]
        x = jnp.concatenate([x2_ref[0], x2_ref[1]], axis=1)
        if si_ref is not None:
            x = x * si_ref[...]
        if pb_ref is not None:
            x = x + pb_ref[...]
        if pre_relu:
            x = jnp.maximum(x, 0.0)
        acc = jnp.dot(x, w_ref[0], preferred_element_type=jnp.float32)
        acc = acc + b_ref[0]
        if relu:
            acc = jnp.maximum(acc, 0.0)
        if so_ref is not None:
            acc = acc * so_ref[...]
        o_ref[0] = acc

    in_specs = [
        pl.BlockSpec((2, BN, dhi), lambda i, j: (0, i, 0)),
        pl.BlockSpec((1, din, dho), lambda i, j: (j, 0, 0)),
        pl.BlockSpec((1, 1, dho), lambda i, j: (j, 0, 0)),
    ]
    args = [h2, W2, b2]
    if have_si:
        in_specs.append(pl.BlockSpec((BN, 1), lambda i, j: (i, 0)))
        args.append(scale_in.reshape(n, 1))
    if have_pb:
        in_specs.append(pl.BlockSpec((1, din), lambda i, j: (0, 0)))
        args.append(pre_bias.reshape(1, din))
    if have_so:
        in_specs.append(pl.BlockSpec((BN, 1), lambda i, j: (i, 0)))
        args.append(scale_out.reshape(n, 1))
    return pl.pallas_call(
        body,
        grid=(n // BN, 2),
        in_specs=in_specs,
        out_specs=pl.BlockSpec((1, BN, dho), lambda i, j: (j, i, 0)),
        out_shape=jax.ShapeDtypeStruct((2, n, dho), jnp.float32),
    )(*args)


def _scale_split(x, dis):
    n, d = x.shape
    dh = d // 2

    def body(x_ref, s_ref, o_ref):
        xs = x_ref[...] * s_ref[...]
        o_ref[0] = xs[:, :dh]
        o_ref[1] = xs[:, dh:]

    return pl.pallas_call(
        body,
        grid=(n // BN,),
        in_specs=[
            pl.BlockSpec((BN, d), lambda i: (i, 0)),
            pl.BlockSpec((BN, 1), lambda i: (i, 0)),
        ],
        out_specs=pl.BlockSpec((2, BN, dh), lambda i: (0, i, 0)),
        out_shape=jax.ShapeDtypeStruct((2, n, dh), jnp.float32),
    )(x, dis.reshape(n, 1))


def _eltwise_split(v2, dis, b, *, relu, dis_out):
    _, n, dh = v2.shape

    def body(v_ref, s_ref, b_ref, o_ref):
        s = s_ref[...]
        for c in range(2):
            y = v_ref[c] * s + b_ref[c]
            if relu:
                y = jnp.maximum(y, 0.0)
            if dis_out:
                y = y * s
            o_ref[c] = y

    return pl.pallas_call(
        body,
        grid=(n // BN,),
        in_specs=[
            pl.BlockSpec((2, BN, dh), lambda i: (0, i, 0)),
            pl.BlockSpec((BN, 1), lambda i: (i, 0)),
            pl.BlockSpec((2, 1, dh), lambda i: (0, 0, 0)),
        ],
        out_specs=pl.BlockSpec((2, BN, dh), lambda i: (0, i, 0)),
        out_shape=jax.ShapeDtypeStruct((2, n, dh), jnp.float32),
    )(v2, dis.reshape(n, 1), b.reshape(2, 1, dh))


def _merge_final(v2, dis, b):
    _, n, dh = v2.shape

    def body(v_ref, s_ref, b_ref, o_ref):
        y = jnp.concatenate([v_ref[0], v_ref[1]], axis=1)
        o_ref[...] = y * s_ref[...] + b_ref[...]

    return pl.pallas_call(
        body,
        grid=(n // BN,),
        in_specs=[
            pl.BlockSpec((2, BN, dh), lambda i: (0, i, 0)),
            pl.BlockSpec((BN, 1), lambda i: (i, 0)),
            pl.BlockSpec((1, 2 * dh), lambda i: (0, 0)),
        ],
        out_specs=pl.BlockSpec((BN, 2 * dh), lambda i: (i, 0)),
        out_shape=jax.ShapeDtypeStruct((n, 2 * dh), jnp.float32),
    )(v2, dis.reshape(n, 1), b.reshape(1, 2 * dh))


def _sc_agg(u2, srcx, dstr, *, dh):
    u2f = u2.reshape(2 * NP, dh)
    mesh = plsc.VectorSubcoreMesh(core_axis_name="c", subcore_axis_name="s")

    @functools.partial(
        pl.kernel,
        out_type=jax.ShapeDtypeStruct((2, NP, dh), jnp.float32),
        mesh=mesh,
        compiler_params=pltpu.CompilerParams(use_tc_tiling_on_sc=False),
        scratch_types=[
            pltpu.MemorySpace.VMEM_SHARED((NP, dh), jnp.float32),
            pltpu.MemorySpace.VMEM((EC,), jnp.int32),
            pltpu.MemorySpace.VMEM((EC,), jnp.int32),
            pltpu.MemorySpace.VMEM((EC, dh), jnp.float32),
            pltpu.SemaphoreType.DMA,
        ],
    )
    def agg(u_hbm, srcx_hbm, dst_hbm, out_hbm,
            acc, src_v, dst_v, rows_v, gsem):
        cc = lax.axis_index("c")
        sid = lax.axis_index("s")
        row0 = sid * RPT

        pltpu.sync_copy(u_hbm.at[pl.ds(cc * NP + row0, RPT)],
                        acc.at[pl.ds(row0, RPT)])
        plsc.subcore_barrier()

        def chunk(g, carry):
            pltpu.sync_copy(srcx_hbm.at[cc, sid, g], src_v)
            pltpu.sync_copy(dst_hbm.at[sid, g], dst_v)
            pltpu.async_copy(u_hbm.at[src_v], rows_v, gsem).wait()
            pltpu.sync_copy(rows_v, acc.at[dst_v], add=True)
            return carry

        lax.fori_loop(0, NCH, chunk, 0)
        plsc.subcore_barrier()

        pltpu.sync_copy(acc.at[pl.ds(row0, RPT)],
                        out_hbm.at[cc, pl.ds(row0, RPT)])

    return agg(u2f, srcx, dstr)


def kernel(x, edge_index, W1, b1, W2, b2, W3, b3, W4, b4, W5, b5,
           W6, b6, W7, b7, W8, b8, W9, b9, W10, b10):
    src = edge_index[0].astype(jnp.int32)
    dst = edge_index[1].astype(jnp.int32)

    ones = jnp.ones((N_EDGES,), dtype=jnp.float32)
    deg = jnp.zeros((N_NODES,), dtype=jnp.float32).at[dst].add(ones) + 1.0
    dis = deg ** -0.5
    dis_pad = jnp.concatenate([dis, jnp.ones((NP - N_NODES,), jnp.float32)])

    srcx = jnp.stack([src, src + NP]).reshape(2, NS, NCH, EC)
    dstr = dst.reshape(NS, NCH, EC)

    x_pad = jnp.concatenate(
        [x, jnp.zeros((NP - N_NODES, x.shape[1]), jnp.float32)])

    u1 = _scale_split(x_pad, dis_pad)
    v1 = _sc_agg(u1, srcx, dstr, dh=64)
    h1 = _dense_split(v1, W1, b1, relu=True, scale_in=dis_pad)

    t2 = _dense_split(h1, W2, jnp.zeros_like(b2), relu=False,
                      scale_out=dis_pad)
    v2 = _sc_agg(t2, srcx, dstr, dh=128)
    t3 = _dense_split(v2, W3, jnp.zeros_like(b3), relu=False,
                      scale_in=dis_pad, pre_bias=b2, pre_relu=True,
                      scale_out=dis_pad)
    v3 = _sc_agg(t3, srcx, dstr, dh=64)
    t4 = _dense_split(v3, W4, jnp.zeros_like(b4), relu=False,
                      scale_in=dis_pad, pre_bias=b3, pre_relu=True,
                      scale_out=dis_pad)
    v4 = _sc_agg(t4, srcx, dstr, dh=32)
    t5 = _dense_split(v4, W5, jnp.zeros_like(b5), relu=False,
                      scale_in=dis_pad, pre_bias=b4, pre_relu=True,
                      scale_out=dis_pad)
    v5 = _sc_agg(t5, srcx, dstr, dh=16)
    u6 = _eltwise_split(v5, dis_pad, b5, relu=True, dis_out=True)
    v6 = _sc_agg(u6, srcx, dstr, dh=16)
    u7 = _dense_split(v6, W6, b6, relu=True, scale_in=dis_pad,
                      scale_out=dis_pad)
    v7 = _sc_agg(u7, srcx, dstr, dh=32)
    u8 = _dense_split(v7, W7, b7, relu=True, scale_in=dis_pad,
                      scale_out=dis_pad)
    v8 = _sc_agg(u8, srcx, dstr, dh=64)
    u9 = _dense_split(v8, W8, b8, relu=True, scale_in=dis_pad,
                      scale_out=dis_pad)
    v9 = _sc_agg(u9, srcx, dstr, dh=128)
    h9 = _dense_split(v9, W9, b9, relu=True, scale_in=dis_pad)
    t10 = _dense_split(h9, W10, jnp.zeros_like(b10), relu=False,
                       scale_out=dis_pad)
    v10 = _sc_agg(t10, srcx, dstr, dh=64)
    out = _merge_final(v10, dis_pad, b10)
    return out[:N_NODES]


# speedup vs baseline: 10.8900x; 2.2167x over previous
import functools

import jax
import jax.numpy as jnp
from jax import lax
from jax.experimental import pallas as pl
from jax.experimental.pallas import tpu as pltpu
from jax.experimental.pallas import tpu_sc as plsc

N_NODES = 10000
N_EDGES = 320000
NP = 10240
BN = 512
NS = 16
NC = 2
RPT = NP // NS
EC = 125
NCH = N_EDGES // NS // EC
KB = 10
NBLK = NCH // KB


def _dense_split(h2, W, b, *, relu, scale_in=None, pre_bias=None,
                 pre_relu=False, scale_out=None):
    _, n, dhi = h2.shape
    din, dout = W.shape
    dho = dout // 2
    W2 = W.reshape(din, 2, dho).transpose(1, 0, 2)
    b2 = b.reshape(2, 1, dho)
    have_si = scale_in is not None
    have_pb = pre_bias is not None
    have_so = scale_out is not None

    def body(*refs):
        x2_ref, w_ref, b_ref = refs[0], refs[1], refs[2]
        k = 3
        si_ref = pb_ref = so_ref = None
        if have_si:
            si_ref = refs[k]; k += 1
        if have_pb:
            pb_ref = refs[k]; k += 1
        if have_so:
            so_ref = refs[k]; k += 1
        o_ref = refs[-1]
        x = jnp.concatenate([x2_ref[0], x2_ref[1]], axis=1)
        if si_ref is not None:
            x = x * si_ref[...]
        if pb_ref is not None:
            x = x + pb_ref[...]
        if pre_relu:
            x = jnp.maximum(x, 0.0)
        acc = jnp.dot(x, w_ref[0], preferred_element_type=jnp.float32)
        acc = acc + b_ref[0]
        if relu:
            acc = jnp.maximum(acc, 0.0)
        if so_ref is not None:
            acc = acc * so_ref[...]
        o_ref[0] = acc

    in_specs = [
        pl.BlockSpec((2, BN, dhi), lambda i, j: (0, i, 0)),
        pl.BlockSpec((1, din, dho), lambda i, j: (j, 0, 0)),
        pl.BlockSpec((1, 1, dho), lambda i, j: (j, 0, 0)),
    ]
    args = [h2, W2, b2]
    if have_si:
        in_specs.append(pl.BlockSpec((BN, 1), lambda i, j: (i, 0)))
        args.append(scale_in.reshape(n, 1))
    if have_pb:
        in_specs.append(pl.BlockSpec((1, din), lambda i, j: (0, 0)))
        args.append(pre_bias.reshape(1, din))
    if have_so:
        in_specs.append(pl.BlockSpec((BN, 1), lambda i, j: (i, 0)))
        args.append(scale_out.reshape(n, 1))
    return pl.pallas_call(
        body,
        grid=(n // BN, 2),
        in_specs=in_specs,
        out_specs=pl.BlockSpec((1, BN, dho), lambda i, j: (j, i, 0)),
        out_shape=jax.ShapeDtypeStruct((2, n, dho), jnp.float32),
    )(*args)


def _scale_split(x, dis):
    n, d = x.shape
    dh = d // 2

    def body(x_ref, s_ref, o_ref):
        xs = x_ref[...] * s_ref[...]
        o_ref[0] = xs[:, :dh]
        o_ref[1] = xs[:, dh:]

    return pl.pallas_call(
        body,
        grid=(n // BN,),
        in_specs=[
            pl.BlockSpec((BN, d), lambda i: (i, 0)),
            pl.BlockSpec((BN, 1), lambda i: (i, 0)),
        ],
        out_specs=pl.BlockSpec((2, BN, dh), lambda i: (0, i, 0)),
        out_shape=jax.ShapeDtypeStruct((2, n, dh), jnp.float32),
    )(x, dis.reshape(n, 1))


def _eltwise_split(v2, dis, b, *, relu, dis_out):
    _, n, dh = v2.shape

    def body(v_ref, s_ref, b_ref, o_ref):
        s = s_ref[...]
        for c in range(2):
            y = v_ref[c] * s + b_ref[c]
            if relu:
                y = jnp.maximum(y, 0.0)
            if dis_out:
                y = y * s
            o_ref[c] = y

    return pl.pallas_call(
        body,
        grid=(n // BN,),
        in_specs=[
            pl.BlockSpec((2, BN, dh), lambda i: (0, i, 0)),
            pl.BlockSpec((BN, 1), lambda i: (i, 0)),
            pl.BlockSpec((2, 1, dh), lambda i: (0, 0, 0)),
        ],
        out_specs=pl.BlockSpec((2, BN, dh), lambda i: (0, i, 0)),
        out_shape=jax.ShapeDtypeStruct((2, n, dh), jnp.float32),
    )(v2, dis.reshape(n, 1), b.reshape(2, 1, dh))


def _merge_final(v2, dis, b):
    _, n, dh = v2.shape

    def body(v_ref, s_ref, b_ref, o_ref):
        y = jnp.concatenate([v_ref[0], v_ref[1]], axis=1)
        o_ref[...] = y * s_ref[...] + b_ref[...]

    return pl.pallas_call(
        body,
        grid=(n // BN,),
        in_specs=[
            pl.BlockSpec((2, BN, dh), lambda i: (0, i, 0)),
            pl.BlockSpec((BN, 1), lambda i: (i, 0)),
            pl.BlockSpec((1, 2 * dh), lambda i: (0, 0)),
        ],
        out_specs=pl.BlockSpec((BN, 2 * dh), lambda i: (i, 0)),
        out_shape=jax.ShapeDtypeStruct((n, 2 * dh), jnp.float32),
    )(v2, dis.reshape(n, 1), b.reshape(1, 2 * dh))


def _sc_agg(u2, srcx, dstr, *, dh):
    u2f = u2.reshape(2 * NP, dh)
    mesh = plsc.VectorSubcoreMesh(core_axis_name="c", subcore_axis_name="s")

    @functools.partial(
        pl.kernel,
        out_type=jax.ShapeDtypeStruct((2, NP, dh), jnp.float32),
        mesh=mesh,
        compiler_params=pltpu.CompilerParams(use_tc_tiling_on_sc=False),
        scratch_types=[
            pltpu.MemorySpace.VMEM_SHARED((NP, dh), jnp.float32),
            pltpu.MemorySpace.VMEM((2, KB, EC), jnp.int32),
            pltpu.MemorySpace.VMEM((2, KB, EC), jnp.int32),
            pltpu.MemorySpace.VMEM((2, EC, dh), jnp.float32),
            pltpu.SemaphoreType.DMA,
            pltpu.SemaphoreType.DMA,
            pltpu.SemaphoreType.DMA,
            pltpu.SemaphoreType.DMA,
        ],
    )
    def agg(u_hbm, srcx_hbm, dst_hbm, out_hbm,
            acc, sidx, didx, rows, isem0, isem1, gsem0, gsem1):
        cc = lax.axis_index("c")
        sid = lax.axis_index("s")
        row0 = sid * RPT
        isems = (isem0, isem1)
        gsems = (gsem0, gsem1)

        def idx_prefetch(p, bb):
            pltpu.async_copy(srcx_hbm.at[cc, sid, pl.ds(bb * KB, KB)],
                             sidx.at[p], isems[p])
            pltpu.async_copy(dst_hbm.at[sid, pl.ds(bb * KB, KB)],
                             didx.at[p], isems[p])

        def idx_drain(p, bb):
            pltpu.make_async_copy(srcx_hbm.at[cc, sid, pl.ds(bb * KB, KB)],
                                  sidx.at[p], isems[p]).wait()
            pltpu.make_async_copy(dst_hbm.at[sid, pl.ds(bb * KB, KB)],
                                  didx.at[p], isems[p]).wait()

        def gather_start(p, j):
            q = j % 2
            pltpu.async_copy(u_hbm.at[sidx.at[p, j]], rows.at[q], gsems[q])

        def gather_wait(p, j):
            q = j % 2
            pltpu.make_async_copy(u_hbm.at[sidx.at[p, j]], rows.at[q],
                                  gsems[q]).wait()

        idx_prefetch(0, 0)
        idx_prefetch(1, 1)

        pltpu.sync_copy(u_hbm.at[pl.ds(cc * NP + row0, RPT)],
                        acc.at[pl.ds(row0, RPT)])
        plsc.subcore_barrier()

        def blockpair(go, carry):
            for p in (0, 1):
                bb = 2 * go + p
                idx_drain(p, bb)
                gather_start(p, 0)
                for j in range(KB):
                    if j + 1 < KB:
                        gather_start(p, j + 1)
                    gather_wait(p, j)
                    pltpu.sync_copy(rows.at[j % 2], acc.at[didx.at[p, j]],
                                    add=True)

                @pl.when(bb + 2 < NBLK)
                def _():
                    idx_prefetch(p, bb + 2)
            return carry

        lax.fori_loop(0, NBLK // 2, blockpair, 0)
        plsc.subcore_barrier()

        pltpu.sync_copy(acc.at[pl.ds(row0, RPT)],
                        out_hbm.at[cc, pl.ds(row0, RPT)])

    return agg(u2f, srcx, dstr)


def kernel(x, edge_index, W1, b1, W2, b2, W3, b3, W4, b4, W5, b5,
           W6, b6, W7, b7, W8, b8, W9, b9, W10, b10):
    src = edge_index[0].astype(jnp.int32)
    dst = edge_index[1].astype(jnp.int32)

    ones = jnp.ones((N_EDGES,), dtype=jnp.float32)
    deg = jnp.zeros((N_NODES,), dtype=jnp.float32).at[dst].add(ones) + 1.0
    dis = deg ** -0.5
    dis_pad = jnp.concatenate([dis, jnp.ones((NP - N_NODES,), jnp.float32)])

    srcx = jnp.stack([src, src + NP]).reshape(2, NS, NCH, EC)
    dstr = dst.reshape(NS, NCH, EC)

    x_pad = jnp.concatenate(
        [x, jnp.zeros((NP - N_NODES, x.shape[1]), jnp.float32)])

    u1 = _scale_split(x_pad, dis_pad)
    v1 = _sc_agg(u1, srcx, dstr, dh=64)
    h1 = _dense_split(v1, W1, b1, relu=True, scale_in=dis_pad)

    t2 = _dense_split(h1, W2, jnp.zeros_like(b2), relu=False,
                      scale_out=dis_pad)
    v2 = _sc_agg(t2, srcx, dstr, dh=128)
    t3 = _dense_split(v2, W3, jnp.zeros_like(b3), relu=False,
                      scale_in=dis_pad, pre_bias=b2, pre_relu=True,
                      scale_out=dis_pad)
    v3 = _sc_agg(t3, srcx, dstr, dh=64)
    t4 = _dense_split(v3, W4, jnp.zeros_like(b4), relu=False,
                      scale_in=dis_pad, pre_bias=b3, pre_relu=True,
                      scale_out=dis_pad)
    v4 = _sc_agg(t4, srcx, dstr, dh=32)
    t5 = _dense_split(v4, W5, jnp.zeros_like(b5), relu=False,
                      scale_in=dis_pad, pre_bias=b4, pre_relu=True,
                      scale_out=dis_pad)
    v5 = _sc_agg(t5, srcx, dstr, dh=16)
    u6 = _eltwise_split(v5, dis_pad, b5, relu=True, dis_out=True)
    v6 = _sc_agg(u6, srcx, dstr, dh=16)
    u7 = _dense_split(v6, W6, b6, relu=True, scale_in=dis_pad,
                      scale_out=dis_pad)
    v7 = _sc_agg(u7, srcx, dstr, dh=32)
    u8 = _dense_split(v7, W7, b7, relu=True, scale_in=dis_pad,
                      scale_out=dis_pad)
    v8 = _sc_agg(u8, srcx, dstr, dh=64)
    u9 = _dense_split(v8, W8, b8, relu=True, scale_in=dis_pad,
                      scale_out=dis_pad)
    v9 = _sc_agg(u9, srcx, dstr, dh=128)
    h9 = _dense_split(v9, W9, b9, relu=True, scale_in=dis_pad)
    t10 = _dense_split(h9, W10, jnp.zeros_like(b10), relu=False,
                       scale_out=dis_pad)
    v10 = _sc_agg(t10, srcx, dstr, dh=64)
    out = _merge_final(v10, dis_pad, b10)
    return out[:N_NODES]


# speedup vs baseline: 11.3673x; 1.0438x over previous
import functools

import jax
import jax.numpy as jnp
from jax import lax
from jax.experimental import pallas as pl
from jax.experimental.pallas import tpu as pltpu
from jax.experimental.pallas import tpu_sc as plsc

N_NODES = 10000
N_EDGES = 320000
NP = 10240
BN = 512
NS = 16
NC = 2
RPT = NP // NS
EC = 125
NCH = N_EDGES // NS // EC
KB = 10
NBLK = NCH // KB


def _dense_split(h2, W, b, *, relu, scale_in=None, pre_bias=None,
                 pre_relu=False, scale_out=None):
    _, n, dhi = h2.shape
    din, dout = W.shape
    dho = dout // 2
    W2 = W.reshape(din, 2, dho).transpose(1, 0, 2)
    b2 = b.reshape(2, 1, dho)
    have_si = scale_in is not None
    have_pb = pre_bias is not None
    have_so = scale_out is not None

    def body(*refs):
        x2_ref, w_ref, b_ref = refs[0], refs[1], refs[2]
        k = 3
        si_ref = pb_ref = so_ref = None
        if have_si:
            si_ref = refs[k]; k += 1
        if have_pb:
            pb_ref = refs[k]; k += 1
        if have_so:
            so_ref = refs[k]; k += 1
        o_ref = refs[-1]
        x = jnp.concatenate([x2_ref[0], x2_ref[1]], axis=1)
        if si_ref is not None:
            x = x * si_ref[...]
        if pb_ref is not None:
            x = x + pb_ref[...]
        if pre_relu:
            x = jnp.maximum(x, 0.0)
        acc = jnp.dot(x, w_ref[0], preferred_element_type=jnp.float32)
        acc = acc + b_ref[0]
        if relu:
            acc = jnp.maximum(acc, 0.0)
        if so_ref is not None:
            acc = acc * so_ref[...]
        o_ref[0] = acc

    in_specs = [
        pl.BlockSpec((2, BN, dhi), lambda i, j: (0, i, 0)),
        pl.BlockSpec((1, din, dho), lambda i, j: (j, 0, 0)),
        pl.BlockSpec((1, 1, dho), lambda i, j: (j, 0, 0)),
    ]
    args = [h2, W2, b2]
    if have_si:
        in_specs.append(pl.BlockSpec((BN, 1), lambda i, j: (i, 0)))
        args.append(scale_in.reshape(n, 1))
    if have_pb:
        in_specs.append(pl.BlockSpec((1, din), lambda i, j: (0, 0)))
        args.append(pre_bias.reshape(1, din))
    if have_so:
        in_specs.append(pl.BlockSpec((BN, 1), lambda i, j: (i, 0)))
        args.append(scale_out.reshape(n, 1))
    return pl.pallas_call(
        body,
        grid=(n // BN, 2),
        in_specs=in_specs,
        out_specs=pl.BlockSpec((1, BN, dho), lambda i, j: (j, i, 0)),
        out_shape=jax.ShapeDtypeStruct((2, n, dho), jnp.float32),
    )(*args)


def _scale_split(x, dis):
    n, d = x.shape
    dh = d // 2

    def body(x_ref, s_ref, o_ref):
        xs = x_ref[...] * s_ref[...]
        o_ref[0] = xs[:, :dh]
        o_ref[1] = xs[:, dh:]

    return pl.pallas_call(
        body,
        grid=(n // BN,),
        in_specs=[
            pl.BlockSpec((BN, d), lambda i: (i, 0)),
            pl.BlockSpec((BN, 1), lambda i: (i, 0)),
        ],
        out_specs=pl.BlockSpec((2, BN, dh), lambda i: (0, i, 0)),
        out_shape=jax.ShapeDtypeStruct((2, n, dh), jnp.float32),
    )(x, dis.reshape(n, 1))


def _eltwise_split(v2, dis, b, *, relu, dis_out):
    _, n, dh = v2.shape

    def body(v_ref, s_ref, b_ref, o_ref):
        s = s_ref[...]
        for c in range(2):
            y = v_ref[c] * s + b_ref[c]
            if relu:
                y = jnp.maximum(y, 0.0)
            if dis_out:
                y = y * s
            o_ref[c] = y

    return pl.pallas_call(
        body,
        grid=(n // BN,),
        in_specs=[
            pl.BlockSpec((2, BN, dh), lambda i: (0, i, 0)),
            pl.BlockSpec((BN, 1), lambda i: (i, 0)),
            pl.BlockSpec((2, 1, dh), lambda i: (0, 0, 0)),
        ],
        out_specs=pl.BlockSpec((2, BN, dh), lambda i: (0, i, 0)),
        out_shape=jax.ShapeDtypeStruct((2, n, dh), jnp.float32),
    )(v2, dis.reshape(n, 1), b.reshape(2, 1, dh))


def _merge_final(v2, dis, b):
    _, n, dh = v2.shape

    def body(v_ref, s_ref, b_ref, o_ref):
        y = jnp.concatenate([v_ref[0], v_ref[1]], axis=1)
        o_ref[...] = y * s_ref[...] + b_ref[...]

    return pl.pallas_call(
        body,
        grid=(n // BN,),
        in_specs=[
            pl.BlockSpec((2, BN, dh), lambda i: (0, i, 0)),
            pl.BlockSpec((BN, 1), lambda i: (i, 0)),
            pl.BlockSpec((1, 2 * dh), lambda i: (0, 0)),
        ],
        out_specs=pl.BlockSpec((BN, 2 * dh), lambda i: (i, 0)),
        out_shape=jax.ShapeDtypeStruct((n, 2 * dh), jnp.float32),
    )(v2, dis.reshape(n, 1), b.reshape(1, 2 * dh))


def _sc_agg(u2, srcx, dstr, *, dh):
    u2f = u2.reshape(2 * NP, dh)
    mesh = plsc.VectorSubcoreMesh(core_axis_name="c", subcore_axis_name="s")

    @functools.partial(
        pl.kernel,
        out_type=jax.ShapeDtypeStruct((2, NP, dh), jnp.float32),
        mesh=mesh,
        compiler_params=pltpu.CompilerParams(use_tc_tiling_on_sc=False),
        scratch_types=[
            pltpu.MemorySpace.VMEM_SHARED((NP, dh), jnp.float32),
            pltpu.MemorySpace.VMEM((2, KB, EC), jnp.int32),
            pltpu.MemorySpace.VMEM((2, KB, EC), jnp.int32),
            pltpu.MemorySpace.VMEM((2, EC, dh), jnp.float32),
            pltpu.SemaphoreType.DMA,
            pltpu.SemaphoreType.DMA,
            pltpu.SemaphoreType.DMA,
            pltpu.SemaphoreType.DMA,
            pltpu.SemaphoreType.DMA,
            pltpu.SemaphoreType.DMA,
        ],
    )
    def agg(u_hbm, srcx_hbm, dst_hbm, out_hbm,
            acc, sidx, didx, rows,
            isem0, isem1, gsem0, gsem1, ssem0, ssem1):
        cc = lax.axis_index("c")
        sid = lax.axis_index("s")
        row0 = sid * RPT
        isems = (isem0, isem1)
        gsems = (gsem0, gsem1)
        ssems = (ssem0, ssem1)

        def idx_prefetch(p, bb):
            pltpu.async_copy(srcx_hbm.at[cc, sid, pl.ds(bb * KB, KB)],
                             sidx.at[p], isems[p])
            pltpu.async_copy(dst_hbm.at[sid, pl.ds(bb * KB, KB)],
                             didx.at[p], isems[p])

        def idx_drain(p, bb):
            pltpu.make_async_copy(srcx_hbm.at[cc, sid, pl.ds(bb * KB, KB)],
                                  sidx.at[p], isems[p]).wait()
            pltpu.make_async_copy(dst_hbm.at[sid, pl.ds(bb * KB, KB)],
                                  didx.at[p], isems[p]).wait()

        def gather_start(p, j):
            q = j % 2
            pltpu.async_copy(u_hbm.at[sidx.at[p, j]], rows.at[q], gsems[q])

        def gather_wait(p, j):
            q = j % 2
            pltpu.make_async_copy(u_hbm.at[sidx.at[p, j]], rows.at[q],
                                  gsems[q]).wait()

        def scatter_start(p, j):
            q = j % 2
            pltpu.async_copy(rows.at[q], acc.at[didx.at[p, j]], ssems[q],
                             add=True)

        def scatter_wait(q):
            pltpu.make_async_copy(rows.at[q], acc.at[didx.at[0, 0]],
                                  ssems[q]).wait()

        idx_prefetch(0, 0)

        pltpu.sync_copy(u_hbm.at[pl.ds(cc * NP + row0, RPT)],
                        acc.at[pl.ds(row0, RPT)])
        plsc.subcore_barrier()

        def blockpair(go, carry):
            for p in (0, 1):
                bb = 2 * go + p
                idx_drain(p, bb)
                for j in range(KB):
                    q = j % 2
                    if p == 0 and j < 2:
                        @pl.when(go > 0)
                        def _():
                            scatter_wait(q)
                    else:
                        scatter_wait(q)
                    gather_start(p, j)
                    if j >= 1:
                        gather_wait(p, j - 1)
                        scatter_start(p, j - 1)
                    elif p == 1:
                        gather_wait(0, KB - 1)
                        scatter_start(0, KB - 1)
                    else:
                        @pl.when(go > 0)
                        def _():
                            gather_wait(1, KB - 1)
                            scatter_start(1, KB - 1)
                    if j == 1:
                        if p == 0:
                            idx_prefetch(1, bb + 1)
                        else:
                            @pl.when(go < NBLK // 2 - 1)
                            def _():
                                idx_prefetch(0, bb + 1)
            return carry

        lax.fori_loop(0, NBLK // 2, blockpair, 0)
        gather_wait(1, KB - 1)
        scatter_start(1, KB - 1)
        scatter_wait(0)
        scatter_wait(1)
        plsc.subcore_barrier()

        pltpu.sync_copy(acc.at[pl.ds(row0, RPT)],
                        out_hbm.at[cc, pl.ds(row0, RPT)])

    return agg(u2f, srcx, dstr)


def kernel(x, edge_index, W1, b1, W2, b2, W3, b3, W4, b4, W5, b5,
           W6, b6, W7, b7, W8, b8, W9, b9, W10, b10):
    src = edge_index[0].astype(jnp.int32)
    dst = edge_index[1].astype(jnp.int32)

    ones = jnp.ones((N_EDGES,), dtype=jnp.float32)
    deg = jnp.zeros((N_NODES,), dtype=jnp.float32).at[dst].add(ones) + 1.0
    dis = deg ** -0.5
    dis_pad = jnp.concatenate([dis, jnp.ones((NP - N_NODES,), jnp.float32)])

    srcx = jnp.stack([src, src + NP]).reshape(2, NS, NCH, EC)
    dstr = dst.reshape(NS, NCH, EC)

    x_pad = jnp.concatenate(
        [x, jnp.zeros((NP - N_NODES, x.shape[1]), jnp.float32)])

    u1 = _scale_split(x_pad, dis_pad)
    v1 = _sc_agg(u1, srcx, dstr, dh=64)
    h1 = _dense_split(v1, W1, b1, relu=True, scale_in=dis_pad)

    t2 = _dense_split(h1, W2, jnp.zeros_like(b2), relu=False,
                      scale_out=dis_pad)
    v2 = _sc_agg(t2, srcx, dstr, dh=128)
    t3 = _dense_split(v2, W3, jnp.zeros_like(b3), relu=False,
                      scale_in=dis_pad, pre_bias=b2, pre_relu=True,
                      scale_out=dis_pad)
    v3 = _sc_agg(t3, srcx, dstr, dh=64)
    t4 = _dense_split(v3, W4, jnp.zeros_like(b4), relu=False,
                      scale_in=dis_pad, pre_bias=b3, pre_relu=True,
                      scale_out=dis_pad)
    v4 = _sc_agg(t4, srcx, dstr, dh=32)
    t5 = _dense_split(v4, W5, jnp.zeros_like(b5), relu=False,
                      scale_in=dis_pad, pre_bias=b4, pre_relu=True,
                      scale_out=dis_pad)
    v5 = _sc_agg(t5, srcx, dstr, dh=16)
    u6 = _eltwise_split(v5, dis_pad, b5, relu=True, dis_out=True)
    v6 = _sc_agg(u6, srcx, dstr, dh=16)
    u7 = _dense_split(v6, W6, b6, relu=True, scale_in=dis_pad,
                      scale_out=dis_pad)
    v7 = _sc_agg(u7, srcx, dstr, dh=32)
    u8 = _dense_split(v7, W7, b7, relu=True, scale_in=dis_pad,
                      scale_out=dis_pad)
    v8 = _sc_agg(u8, srcx, dstr, dh=64)
    u9 = _dense_split(v8, W8, b8, relu=True, scale_in=dis_pad,
                      scale_out=dis_pad)
    v9 = _sc_agg(u9, srcx, dstr, dh=128)
    h9 = _dense_split(v9, W9, b9, relu=True, scale_in=dis_pad)
    t10 = _dense_split(h9, W10, jnp.zeros_like(b10), relu=False,
                       scale_out=dis_pad)
    v10 = _sc_agg(t10, srcx, dstr, dh=64)
    out = _merge_final(v10, dis_pad, b10)
    return out[:N_NODES]


# speedup vs baseline: 12.5203x; 1.1014x over previous
import functools

import jax
import jax.numpy as jnp
from jax import lax
from jax.experimental import pallas as pl
from jax.experimental.pallas import tpu as pltpu
from jax.experimental.pallas import tpu_sc as plsc

N_NODES = 10000
N_EDGES = 320000
NP = 10240
BN = 512
NS = 16
NC = 2
RPT = NP // NS
EC = 125
NCH = N_EDGES // NS // EC
KB = 10
NBLK = NCH // KB


def _dense_split(h2, W, b, *, relu, scale_in=None, pre_bias=None,
                 pre_relu=False, scale_out=None):
    _, n, dhi = h2.shape
    din, dout = W.shape
    dho = dout // 2
    W2 = W.reshape(din, 2, dho).transpose(1, 0, 2)
    b2 = b.reshape(2, 1, dho)
    have_si = scale_in is not None
    have_pb = pre_bias is not None
    have_so = scale_out is not None

    def body(*refs):
        x2_ref, w_ref, b_ref = refs[0], refs[1], refs[2]
        k = 3
        si_ref = pb_ref = so_ref = None
        if have_si:
            si_ref = refs[k]; k += 1
        if have_pb:
            pb_ref = refs[k]; k += 1
        if have_so:
            so_ref = refs[k]; k += 1
        o_ref = refs[-1]
        x = jnp.concatenate([x2_ref[0], x2_ref[1]], axis=1)
        if si_ref is not None:
            x = x * si_ref[...]
        if pb_ref is not None:
            x = x + pb_ref[...]
        if pre_relu:
            x = jnp.maximum(x, 0.0)
        acc = jnp.dot(x, w_ref[0], preferred_element_type=jnp.float32)
        acc = acc + b_ref[0]
        if relu:
            acc = jnp.maximum(acc, 0.0)
        if so_ref is not None:
            acc = acc * so_ref[...]
        o_ref[0] = acc

    in_specs = [
        pl.BlockSpec((2, BN, dhi), lambda i, j: (0, i, 0)),
        pl.BlockSpec((1, din, dho), lambda i, j: (j, 0, 0)),
        pl.BlockSpec((1, 1, dho), lambda i, j: (j, 0, 0)),
    ]
    args = [h2, W2, b2]
    if have_si:
        in_specs.append(pl.BlockSpec((BN, 1), lambda i, j: (i, 0)))
        args.append(scale_in.reshape(n, 1))
    if have_pb:
        in_specs.append(pl.BlockSpec((1, din), lambda i, j: (0, 0)))
        args.append(pre_bias.reshape(1, din))
    if have_so:
        in_specs.append(pl.BlockSpec((BN, 1), lambda i, j: (i, 0)))
        args.append(scale_out.reshape(n, 1))
    return pl.pallas_call(
        body,
        grid=(n // BN, 2),
        in_specs=in_specs,
        out_specs=pl.BlockSpec((1, BN, dho), lambda i, j: (j, i, 0)),
        out_shape=jax.ShapeDtypeStruct((2, n, dho), jnp.float32),
    )(*args)


def _scale_split(x, dis):
    n, d = x.shape
    dh = d // 2

    def body(x_ref, s_ref, o_ref):
        xs = x_ref[...] * s_ref[...]
        o_ref[0] = xs[:, :dh]
        o_ref[1] = xs[:, dh:]

    return pl.pallas_call(
        body,
        grid=(n // BN,),
        in_specs=[
            pl.BlockSpec((BN, d), lambda i: (i, 0)),
            pl.BlockSpec((BN, 1), lambda i: (i, 0)),
        ],
        out_specs=pl.BlockSpec((2, BN, dh), lambda i: (0, i, 0)),
        out_shape=jax.ShapeDtypeStruct((2, n, dh), jnp.float32),
    )(x, dis.reshape(n, 1))


def _eltwise_split(v2, dis, b, *, relu, dis_out):
    _, n, dh = v2.shape

    def body(v_ref, s_ref, b_ref, o_ref):
        s = s_ref[...]
        for c in range(2):
            y = v_ref[c] * s + b_ref[c]
            if relu:
                y = jnp.maximum(y, 0.0)
            if dis_out:
                y = y * s
            o_ref[c] = y

    return pl.pallas_call(
        body,
        grid=(n // BN,),
        in_specs=[
            pl.BlockSpec((2, BN, dh), lambda i: (0, i, 0)),
            pl.BlockSpec((BN, 1), lambda i: (i, 0)),
            pl.BlockSpec((2, 1, dh), lambda i: (0, 0, 0)),
        ],
        out_specs=pl.BlockSpec((2, BN, dh), lambda i: (0, i, 0)),
        out_shape=jax.ShapeDtypeStruct((2, n, dh), jnp.float32),
    )(v2, dis.reshape(n, 1), b.reshape(2, 1, dh))


def _merge_final(v2, dis, b):
    _, n, dh = v2.shape

    def body(v_ref, s_ref, b_ref, o_ref):
        y = jnp.concatenate([v_ref[0], v_ref[1]], axis=1)
        o_ref[...] = y * s_ref[...] + b_ref[...]

    return pl.pallas_call(
        body,
        grid=(n // BN,),
        in_specs=[
            pl.BlockSpec((2, BN, dh), lambda i: (0, i, 0)),
            pl.BlockSpec((BN, 1), lambda i: (i, 0)),
            pl.BlockSpec((1, 2 * dh), lambda i: (0, 0)),
        ],
        out_specs=pl.BlockSpec((BN, 2 * dh), lambda i: (i, 0)),
        out_shape=jax.ShapeDtypeStruct((n, 2 * dh), jnp.float32),
    )(v2, dis.reshape(n, 1), b.reshape(1, 2 * dh))


DW = 16


def _sc_degree(dstr):
    mesh = plsc.VectorSubcoreMesh(core_axis_name="c", subcore_axis_name="s")
    nchd = NCH // 2
    nblkd = nchd // KB

    @functools.partial(
        pl.kernel,
        out_type=jax.ShapeDtypeStruct((2, NP, DW), jnp.float32),
        mesh=mesh,
        compiler_params=pltpu.CompilerParams(use_tc_tiling_on_sc=False),
        scratch_types=[
            pltpu.MemorySpace.VMEM_SHARED((NP, DW), jnp.float32),
            pltpu.MemorySpace.VMEM((2, KB, EC), jnp.int32),
            pltpu.MemorySpace.VMEM((EC, DW), jnp.float32),
            pltpu.MemorySpace.VMEM((RPT, DW), jnp.float32),
            pltpu.SemaphoreType.DMA,
            pltpu.SemaphoreType.DMA,
        ],
    )
    def deg(dst_hbm, out_hbm, acc, didx, ones_v, zero_v, isem0, isem1):
        cc = lax.axis_index("c")
        sid = lax.axis_index("s")
        row0 = sid * RPT
        isems = (isem0, isem1)

        def idx_prefetch(p, bb):
            pltpu.async_copy(
                dst_hbm.at[sid, pl.ds(cc * nchd + bb * KB, KB)],
                didx.at[p], isems[p])

        def idx_drain(p, bb):
            pltpu.make_async_copy(
                dst_hbm.at[sid, pl.ds(cc * nchd + bb * KB, KB)],
                didx.at[p], isems[p]).wait()

        idx_prefetch(0, 0)

        def fill_ones(r, carry):
            ones_v[r, :] = jnp.ones((DW,), jnp.float32)
            return carry

        def fill_zero(r, carry):
            zero_v[r, :] = jnp.zeros((DW,), jnp.float32)
            return carry

        lax.fori_loop(0, EC, fill_ones, 0)
        lax.fori_loop(0, RPT, fill_zero, 0)
        pltpu.sync_copy(zero_v, acc.at[pl.ds(row0, RPT)])
        plsc.subcore_barrier()

        def blockpair(go, carry):
            for p in (0, 1):
                bb = 2 * go + p
                idx_drain(p, bb)
                if p == 0:
                    idx_prefetch(1, bb + 1)
                else:
                    @pl.when(go < nblkd // 2 - 1)
                    def _():
                        idx_prefetch(0, bb + 1)
                for j in range(KB):
                    pltpu.sync_copy(ones_v, acc.at[didx.at[p, j]], add=True)
            return carry

        lax.fori_loop(0, nblkd // 2, blockpair, 0)
        plsc.subcore_barrier()
        pltpu.sync_copy(acc.at[pl.ds(row0, RPT)],
                        out_hbm.at[cc, pl.ds(row0, RPT)])

    return deg(dstr)


def _sc_agg(u2, srcx, dstr, *, dh):
    u2f = u2.reshape(2 * NP, dh)
    mesh = plsc.VectorSubcoreMesh(core_axis_name="c", subcore_axis_name="s")

    @functools.partial(
        pl.kernel,
        out_type=jax.ShapeDtypeStruct((2, NP, dh), jnp.float32),
        mesh=mesh,
        compiler_params=pltpu.CompilerParams(use_tc_tiling_on_sc=False),
        scratch_types=[
            pltpu.MemorySpace.VMEM_SHARED((NP, dh), jnp.float32),
            pltpu.MemorySpace.VMEM((2, KB, EC), jnp.int32),
            pltpu.MemorySpace.VMEM((2, KB, EC), jnp.int32),
            pltpu.MemorySpace.VMEM((2, EC, dh), jnp.float32),
            pltpu.SemaphoreType.DMA,
            pltpu.SemaphoreType.DMA,
            pltpu.SemaphoreType.DMA,
            pltpu.SemaphoreType.DMA,
            pltpu.SemaphoreType.DMA,
            pltpu.SemaphoreType.DMA,
        ],
    )
    def agg(u_hbm, srcx_hbm, dst_hbm, out_hbm,
            acc, sidx, didx, rows,
            isem0, isem1, gsem0, gsem1, ssem0, ssem1):
        cc = lax.axis_index("c")
        sid = lax.axis_index("s")
        row0 = sid * RPT
        isems = (isem0, isem1)
        gsems = (gsem0, gsem1)
        ssems = (ssem0, ssem1)

        def idx_prefetch(p, bb):
            pltpu.async_copy(srcx_hbm.at[cc, sid, pl.ds(bb * KB, KB)],
                             sidx.at[p], isems[p])
            pltpu.async_copy(dst_hbm.at[sid, pl.ds(bb * KB, KB)],
                             didx.at[p], isems[p])

        def idx_drain(p, bb):
            pltpu.make_async_copy(srcx_hbm.at[cc, sid, pl.ds(bb * KB, KB)],
                                  sidx.at[p], isems[p]).wait()
            pltpu.make_async_copy(dst_hbm.at[sid, pl.ds(bb * KB, KB)],
                                  didx.at[p], isems[p]).wait()

        def gather_start(p, j):
            q = j % 2
            pltpu.async_copy(u_hbm.at[sidx.at[p, j]], rows.at[q], gsems[q])

        def gather_wait(p, j):
            q = j % 2
            pltpu.make_async_copy(u_hbm.at[sidx.at[p, j]], rows.at[q],
                                  gsems[q]).wait()

        def scatter_start(p, j):
            q = j % 2
            pltpu.async_copy(rows.at[q], acc.at[didx.at[p, j]], ssems[q],
                             add=True)

        def scatter_wait(q):
            pltpu.make_async_copy(rows.at[q], acc.at[didx.at[0, 0]],
                                  ssems[q]).wait()

        idx_prefetch(0, 0)

        pltpu.sync_copy(u_hbm.at[pl.ds(cc * NP + row0, RPT)],
                        acc.at[pl.ds(row0, RPT)])
        plsc.subcore_barrier()

        def blockpair(go, carry):
            for p in (0, 1):
                bb = 2 * go + p
                idx_drain(p, bb)
                for j in range(KB):
                    q = j % 2
                    if p == 0 and j < 2:
                        @pl.when(go > 0)
                        def _():
                            scatter_wait(q)
                    else:
                        scatter_wait(q)
                    gather_start(p, j)
                    if j >= 1:
                        gather_wait(p, j - 1)
                        scatter_start(p, j - 1)
                    elif p == 1:
                        gather_wait(0, KB - 1)
                        scatter_start(0, KB - 1)
                    else:
                        @pl.when(go > 0)
                        def _():
                            gather_wait(1, KB - 1)
                            scatter_start(1, KB - 1)
                    if j == 1:
                        if p == 0:
                            idx_prefetch(1, bb + 1)
                        else:
                            @pl.when(go < NBLK // 2 - 1)
                            def _():
                                idx_prefetch(0, bb + 1)
            return carry

        lax.fori_loop(0, NBLK // 2, blockpair, 0)
        gather_wait(1, KB - 1)
        scatter_start(1, KB - 1)
        scatter_wait(0)
        scatter_wait(1)
        plsc.subcore_barrier()

        pltpu.sync_copy(acc.at[pl.ds(row0, RPT)],
                        out_hbm.at[cc, pl.ds(row0, RPT)])

    return agg(u2f, srcx, dstr)


def kernel(x, edge_index, W1, b1, W2, b2, W3, b3, W4, b4, W5, b5,
           W6, b6, W7, b7, W8, b8, W9, b9, W10, b10):
    src = edge_index[0].astype(jnp.int32)
    dst = edge_index[1].astype(jnp.int32)

    srcx = jnp.stack([src, src + NP]).reshape(2, NS, NCH, EC)
    dstr = dst.reshape(NS, NCH, EC)

    cnt = _sc_degree(dstr)
    deg_pad = cnt[0, :, 0] + cnt[1, :, 0] + 1.0
    dis_pad = deg_pad ** -0.5

    x_pad = jnp.concatenate(
        [x, jnp.zeros((NP - N_NODES, x.shape[1]), jnp.float32)])

    u1 = _scale_split(x_pad, dis_pad)
    v1 = _sc_agg(u1, srcx, dstr, dh=64)
    h1 = _dense_split(v1, W1, b1, relu=True, scale_in=dis_pad)

    t2 = _dense_split(h1, W2, jnp.zeros_like(b2), relu=False,
                      scale_out=dis_pad)
    v2 = _sc_agg(t2, srcx, dstr, dh=128)
    t3 = _dense_split(v2, W3, jnp.zeros_like(b3), relu=False,
                      scale_in=dis_pad, pre_bias=b2, pre_relu=True,
                      scale_out=dis_pad)
    v3 = _sc_agg(t3, srcx, dstr, dh=64)
    t4 = _dense_split(v3, W4, jnp.zeros_like(b4), relu=False,
                      scale_in=dis_pad, pre_bias=b3, pre_relu=True,
                      scale_out=dis_pad)
    v4 = _sc_agg(t4, srcx, dstr, dh=32)
    t5 = _dense_split(v4, W5, jnp.zeros_like(b5), relu=False,
                      scale_in=dis_pad, pre_bias=b4, pre_relu=True,
                      scale_out=dis_pad)
    v5 = _sc_agg(t5, srcx, dstr, dh=16)
    u6 = _eltwise_split(v5, dis_pad, b5, relu=True, dis_out=True)
    v6 = _sc_agg(u6, srcx, dstr, dh=16)
    u7 = _dense_split(v6, W6, b6, relu=True, scale_in=dis_pad,
                      scale_out=dis_pad)
    v7 = _sc_agg(u7, srcx, dstr, dh=32)
    u8 = _dense_split(v7, W7, b7, relu=True, scale_in=dis_pad,
                      scale_out=dis_pad)
    v8 = _sc_agg(u8, srcx, dstr, dh=64)
    u9 = _dense_split(v8, W8, b8, relu=True, scale_in=dis_pad,
                      scale_out=dis_pad)
    v9 = _sc_agg(u9, srcx, dstr, dh=128)
    h9 = _dense_split(v9, W9, b9, relu=True, scale_in=dis_pad)
    t10 = _dense_split(h9, W10, jnp.zeros_like(b10), relu=False,
                       scale_out=dis_pad)
    v10 = _sc_agg(t10, srcx, dstr, dh=64)
    out = _merge_final(v10, dis_pad, b10)
    return out[:N_NODES]


# speedup vs baseline: 15.8021x; 1.2621x over previous
import functools

import jax
import jax.numpy as jnp
from jax import lax
from jax.experimental import pallas as pl
from jax.experimental.pallas import tpu as pltpu
from jax.experimental.pallas import tpu_sc as plsc

N_NODES = 10000
N_EDGES = 320000
NP = 10240
BN = 512
NS = 16
NC = 2
RPT = NP // NS
EC = 125
NCH = N_EDGES // NS // EC
KB = 10
NBLK = NCH // KB


def _dense_split(h2, W, b, *, relu, scale_in=None, pre_bias=None,
                 pre_relu=False, scale_out=None):
    _, n, dhi = h2.shape
    din, dout = W.shape
    dho = dout // 2
    W2 = W.reshape(din, 2, dho).transpose(1, 0, 2)
    b2 = b.reshape(2, 1, dho)
    have_si = scale_in is not None
    have_pb = pre_bias is not None
    have_so = scale_out is not None

    def body(*refs):
        x2_ref, w_ref, b_ref = refs[0], refs[1], refs[2]
        k = 3
        si_ref = pb_ref = so_ref = None
        if have_si:
            si_ref = refs[k]; k += 1
        if have_pb:
            pb_ref = refs[k]; k += 1
        if have_so:
            so_ref = refs[k]; k += 1
        o_ref = refs[-1]
        x = jnp.concatenate([x2_ref[0], x2_ref[1]], axis=1)
        if si_ref is not None:
            x = x * si_ref[...]
        if pb_ref is not None:
            x = x + pb_ref[...]
        if pre_relu:
            x = jnp.maximum(x, 0.0)
        acc = jnp.dot(x, w_ref[0], preferred_element_type=jnp.float32)
        acc = acc + b_ref[0]
        if relu:
            acc = jnp.maximum(acc, 0.0)
        if so_ref is not None:
            acc = acc * so_ref[...]
        o_ref[0] = acc

    in_specs = [
        pl.BlockSpec((2, BN, dhi), lambda i, j: (0, i, 0)),
        pl.BlockSpec((1, din, dho), lambda i, j: (j, 0, 0)),
        pl.BlockSpec((1, 1, dho), lambda i, j: (j, 0, 0)),
    ]
    args = [h2, W2, b2]
    if have_si:
        in_specs.append(pl.BlockSpec((BN, 1), lambda i, j: (i, 0)))
        args.append(scale_in.reshape(n, 1))
    if have_pb:
        in_specs.append(pl.BlockSpec((1, din), lambda i, j: (0, 0)))
        args.append(pre_bias.reshape(1, din))
    if have_so:
        in_specs.append(pl.BlockSpec((BN, 1), lambda i, j: (i, 0)))
        args.append(scale_out.reshape(n, 1))
    return pl.pallas_call(
        body,
        grid=(n // BN, 2),
        in_specs=in_specs,
        out_specs=pl.BlockSpec((1, BN, dho), lambda i, j: (j, i, 0)),
        out_shape=jax.ShapeDtypeStruct((2, n, dho), jnp.float32),
    )(*args)


def _scale_split(x, dis):
    n, d = x.shape
    dh = d // 2

    def body(x_ref, s_ref, o_ref):
        xs = x_ref[...] * s_ref[...]
        o_ref[0] = xs[:, :dh]
        o_ref[1] = xs[:, dh:]

    return pl.pallas_call(
        body,
        grid=(n // BN,),
        in_specs=[
            pl.BlockSpec((BN, d), lambda i: (i, 0)),
            pl.BlockSpec((BN, 1), lambda i: (i, 0)),
        ],
        out_specs=pl.BlockSpec((2, BN, dh), lambda i: (0, i, 0)),
        out_shape=jax.ShapeDtypeStruct((2, n, dh), jnp.float32),
    )(x, dis.reshape(n, 1))


def _eltwise_split(v2, dis, b, *, relu, dis_out):
    _, n, dh = v2.shape

    def body(v_ref, s_ref, b_ref, o_ref):
        s = s_ref[...]
        for c in range(2):
            y = v_ref[c] * s + b_ref[c]
            if relu:
                y = jnp.maximum(y, 0.0)
            if dis_out:
                y = y * s
            o_ref[c] = y

    return pl.pallas_call(
        body,
        grid=(n // BN,),
        in_specs=[
            pl.BlockSpec((2, BN, dh), lambda i: (0, i, 0)),
            pl.BlockSpec((BN, 1), lambda i: (i, 0)),
            pl.BlockSpec((2, 1, dh), lambda i: (0, 0, 0)),
        ],
        out_specs=pl.BlockSpec((2, BN, dh), lambda i: (0, i, 0)),
        out_shape=jax.ShapeDtypeStruct((2, n, dh), jnp.float32),
    )(v2, dis.reshape(n, 1), b.reshape(2, 1, dh))


def _merge_final(v2, dis, b):
    _, n, dh = v2.shape

    def body(v_ref, s_ref, b_ref, o_ref):
        y = jnp.concatenate([v_ref[0], v_ref[1]], axis=1)
        o_ref[...] = y * s_ref[...] + b_ref[...]

    return pl.pallas_call(
        body,
        grid=(n // BN,),
        in_specs=[
            pl.BlockSpec((2, BN, dh), lambda i: (0, i, 0)),
            pl.BlockSpec((BN, 1), lambda i: (i, 0)),
            pl.BlockSpec((1, 2 * dh), lambda i: (0, 0)),
        ],
        out_specs=pl.BlockSpec((BN, 2 * dh), lambda i: (i, 0)),
        out_shape=jax.ShapeDtypeStruct((n, 2 * dh), jnp.float32),
    )(v2, dis.reshape(n, 1), b.reshape(1, 2 * dh))


DW = 16


def _sc_degree(dstr):
    mesh = plsc.VectorSubcoreMesh(core_axis_name="c", subcore_axis_name="s")
    nchd = NCH // 2
    nblkd = nchd // KB

    @functools.partial(
        pl.kernel,
        out_type=jax.ShapeDtypeStruct((2, NP, DW), jnp.float32),
        mesh=mesh,
        compiler_params=pltpu.CompilerParams(use_tc_tiling_on_sc=False),
        scratch_types=[
            pltpu.MemorySpace.VMEM_SHARED((NP, DW), jnp.float32),
            pltpu.MemorySpace.VMEM((2, KB, EC), jnp.int32),
            pltpu.MemorySpace.VMEM((EC, DW), jnp.float32),
            pltpu.MemorySpace.VMEM((RPT, DW), jnp.float32),
            pltpu.SemaphoreType.DMA,
            pltpu.SemaphoreType.DMA,
        ],
    )
    def deg(dst_hbm, out_hbm, acc, didx, ones_v, zero_v, isem0, isem1):
        cc = lax.axis_index("c")
        sid = lax.axis_index("s")
        row0 = sid * RPT
        isems = (isem0, isem1)

        def idx_prefetch(p, bb):
            pltpu.async_copy(
                dst_hbm.at[sid, pl.ds(cc * nchd + bb * KB, KB)],
                didx.at[p], isems[p])

        def idx_drain(p, bb):
            pltpu.make_async_copy(
                dst_hbm.at[sid, pl.ds(cc * nchd + bb * KB, KB)],
                didx.at[p], isems[p]).wait()

        idx_prefetch(0, 0)

        def fill_ones(r, carry):
            ones_v[r, :] = jnp.ones((DW,), jnp.float32)
            return carry

        def fill_zero(r, carry):
            zero_v[r, :] = jnp.zeros((DW,), jnp.float32)
            return carry

        lax.fori_loop(0, EC, fill_ones, 0)
        lax.fori_loop(0, RPT, fill_zero, 0)
        pltpu.sync_copy(zero_v, acc.at[pl.ds(row0, RPT)])
        plsc.subcore_barrier()

        def blockpair(go, carry):
            for p in (0, 1):
                bb = 2 * go + p
                idx_drain(p, bb)
                if p == 0:
                    idx_prefetch(1, bb + 1)
                else:
                    @pl.when(go < nblkd // 2 - 1)
                    def _():
                        idx_prefetch(0, bb + 1)
                for j in range(KB):
                    pltpu.sync_copy(ones_v, acc.at[didx.at[p, j]], add=True)
            return carry

        lax.fori_loop(0, nblkd // 2, blockpair, 0)
        plsc.subcore_barrier()
        pltpu.sync_copy(acc.at[pl.ds(row0, RPT)],
                        out_hbm.at[cc, pl.ds(row0, RPT)])

    return deg(dstr)


def _sc_agg(u2, srcx, dstr, *, dh):
    u2f = u2.reshape(2 * NP, dh)
    mesh = plsc.VectorSubcoreMesh(core_axis_name="c", subcore_axis_name="s")
    ring = 2 if dh == 128 else 5

    sems = [pltpu.SemaphoreType.DMA] * (2 + 2 * ring)

    @functools.partial(
        pl.kernel,
        out_type=jax.ShapeDtypeStruct((2, NP, dh), jnp.float32),
        mesh=mesh,
        compiler_params=pltpu.CompilerParams(use_tc_tiling_on_sc=False),
        scratch_types=[
            pltpu.MemorySpace.VMEM_SHARED((NP, dh), jnp.float32),
            pltpu.MemorySpace.VMEM((2, KB, EC), jnp.int32),
            pltpu.MemorySpace.VMEM((2, KB, EC), jnp.int32),
            pltpu.MemorySpace.VMEM((ring, EC, dh), jnp.float32),
        ] + sems,
    )
    def agg(*refs):
        (u_hbm, srcx_hbm, dst_hbm, out_hbm, acc, sidx, didx, rows) = refs[:8]
        isems = refs[8:10]
        gsems = refs[10:10 + ring]
        ssems = refs[10 + ring:10 + 2 * ring]
        cc = lax.axis_index("c")
        sid = lax.axis_index("s")
        row0 = sid * RPT

        def idx_prefetch(p, bb):
            pltpu.async_copy(srcx_hbm.at[cc, sid, pl.ds(bb * KB, KB)],
                             sidx.at[p], isems[p])
            pltpu.async_copy(dst_hbm.at[sid, pl.ds(bb * KB, KB)],
                             didx.at[p], isems[p])

        def idx_drain(p, bb):
            pltpu.make_async_copy(srcx_hbm.at[cc, sid, pl.ds(bb * KB, KB)],
                                  sidx.at[p], isems[p]).wait()
            pltpu.make_async_copy(dst_hbm.at[sid, pl.ds(bb * KB, KB)],
                                  didx.at[p], isems[p]).wait()

        def gather_start(p, j):
            q = j % ring
            pltpu.async_copy(u_hbm.at[sidx.at[p, j]], rows.at[q], gsems[q])

        def gather_wait(p, j):
            q = j % ring
            pltpu.make_async_copy(u_hbm.at[sidx.at[p, j]], rows.at[q],
                                  gsems[q]).wait()

        def scatter_start(p, j):
            q = j % ring
            pltpu.async_copy(rows.at[q], acc.at[didx.at[p, j]], ssems[q],
                             add=True)

        def scatter_wait(q):
            pltpu.make_async_copy(rows.at[q], acc.at[didx.at[0, 0]],
                                  ssems[q]).wait()

        idx_prefetch(0, 0)

        pltpu.sync_copy(u_hbm.at[pl.ds(cc * NP + row0, RPT)],
                        acc.at[pl.ds(row0, RPT)])
        plsc.subcore_barrier()

        G = 1 if ring == 2 else 3

        def blockpair(go, carry):
            for p in (0, 1):
                bb = 2 * go + p
                idx_drain(p, bb)
                for j in range(KB):
                    q = j % ring
                    if p == 0 and j < ring:
                        @pl.when(go > 0)
                        def _():
                            scatter_wait(q)
                    else:
                        scatter_wait(q)
                    gather_start(p, j)
                    if j >= G:
                        gather_wait(p, j - G)
                        scatter_start(p, j - G)
                    elif p == 1:
                        gather_wait(0, KB - G + j)
                        scatter_start(0, KB - G + j)
                    else:
                        @pl.when(go > 0)
                        def _():
                            gather_wait(1, KB - G + j)
                            scatter_start(1, KB - G + j)
                    if j == ring - 1:
                        if p == 0:
                            idx_prefetch(1, bb + 1)
                        else:
                            @pl.when(go < NBLK // 2 - 1)
                            def _():
                                idx_prefetch(0, bb + 1)
            return carry

        lax.fori_loop(0, NBLK // 2, blockpair, 0)
        for t in range(G):
            gather_wait(1, KB - G + t)
            scatter_start(1, KB - G + t)
        for q in range(ring):
            scatter_wait(q)
        plsc.subcore_barrier()

        pltpu.sync_copy(acc.at[pl.ds(row0, RPT)],
                        out_hbm.at[cc, pl.ds(row0, RPT)])

    return agg(u2f, srcx, dstr)


def kernel(x, edge_index, W1, b1, W2, b2, W3, b3, W4, b4, W5, b5,
           W6, b6, W7, b7, W8, b8, W9, b9, W10, b10):
    src = edge_index[0].astype(jnp.int32)
    dst = edge_index[1].astype(jnp.int32)

    srcx = jnp.stack([src, src + NP]).reshape(2, NS, NCH, EC)
    dstr = dst.reshape(NS, NCH, EC)

    cnt = _sc_degree(dstr)
    deg_pad = cnt[0, :, 0] + cnt[1, :, 0] + 1.0
    dis_pad = deg_pad ** -0.5

    x_pad = jnp.concatenate(
        [x, jnp.zeros((NP - N_NODES, x.shape[1]), jnp.float32)])

    u1 = _scale_split(x_pad, dis_pad)
    v1 = _sc_agg(u1, srcx, dstr, dh=64)
    h1 = _dense_split(v1, W1, b1, relu=True, scale_in=dis_pad)

    t2 = _dense_split(h1, W2, jnp.zeros_like(b2), relu=False,
                      scale_out=dis_pad)
    v2 = _sc_agg(t2, srcx, dstr, dh=128)
    t3 = _dense_split(v2, W3, jnp.zeros_like(b3), relu=False,
                      scale_in=dis_pad, pre_bias=b2, pre_relu=True,
                      scale_out=dis_pad)
    v3 = _sc_agg(t3, srcx, dstr, dh=64)
    t4 = _dense_split(v3, W4, jnp.zeros_like(b4), relu=False,
                      scale_in=dis_pad, pre_bias=b3, pre_relu=True,
                      scale_out=dis_pad)
    v4 = _sc_agg(t4, srcx, dstr, dh=32)
    t5 = _dense_split(v4, W5, jnp.zeros_like(b5), relu=False,
                      scale_in=dis_pad, pre_bias=b4, pre_relu=True,
                      scale_out=dis_pad)
    v5 = _sc_agg(t5, srcx, dstr, dh=16)
    u6 = _eltwise_split(v5, dis_pad, b5, relu=True, dis_out=True)
    v6 = _sc_agg(u6, srcx, dstr, dh=16)
    u7 = _dense_split(v6, W6, b6, relu=True, scale_in=dis_pad,
                      scale_out=dis_pad)
    v7 = _sc_agg(u7, srcx, dstr, dh=32)
    u8 = _dense_split(v7, W7, b7, relu=True, scale_in=dis_pad,
                      scale_out=dis_pad)
    v8 = _sc_agg(u8, srcx, dstr, dh=64)
    u9 = _dense_split(v8, W8, b8, relu=True, scale_in=dis_pad,
                      scale_out=dis_pad)
    v9 = _sc_agg(u9, srcx, dstr, dh=128)
    h9 = _dense_split(v9, W9, b9, relu=True, scale_in=dis_pad)
    t10 = _dense_split(h9, W10, jnp.zeros_like(b10), relu=False,
                       scale_out=dis_pad)
    v10 = _sc_agg(t10, srcx, dstr, dh=64)
    out = _merge_final(v10, dis_pad, b10)
    return out[:N_NODES]


# speedup vs baseline: 17.8739x; 1.1311x over previous
import functools

import jax
import jax.numpy as jnp
from jax import lax
from jax.experimental import pallas as pl
from jax.experimental.pallas import tpu as pltpu
from jax.experimental.pallas import tpu_sc as plsc

N_NODES = 10000
N_EDGES = 320000
NP = 10240
BN = 512
NS = 16
NC = 2
RPT = NP // NS
EC = 125
NCH = N_EDGES // NS // EC
KB = 10
NBLK = NCH // KB


def _dense_split(h2, W, b, *, relu, scale_in=None, pre_bias=None,
                 pre_relu=False, scale_out=None):
    _, n, dhi = h2.shape
    din, dout = W.shape
    dho = dout // 2
    b2 = b.reshape(1, dout)
    have_si = scale_in is not None
    have_pb = pre_bias is not None
    have_so = scale_out is not None

    def body(*refs):
        x2_ref, w_ref, b_ref = refs[0], refs[1], refs[2]
        k = 3
        si_ref = pb_ref = so_ref = None
        if have_si:
            si_ref = refs[k]; k += 1
        if have_pb:
            pb_ref = refs[k]; k += 1
        if have_so:
            so_ref = refs[k]; k += 1
        o_ref = refs[-1]
        x = jnp.concatenate([x2_ref[0], x2_ref[1]], axis=1)
        if si_ref is not None:
            x = x * si_ref[...]
        if pb_ref is not None:
            x = x + pb_ref[...]
        if pre_relu:
            x = jnp.maximum(x, 0.0)
        acc = jnp.dot(x, w_ref[...], preferred_element_type=jnp.float32)
        acc = acc + b_ref[...]
        if relu:
            acc = jnp.maximum(acc, 0.0)
        if so_ref is not None:
            acc = acc * so_ref[...]
        o_ref[0] = acc[:, :dho]
        o_ref[1] = acc[:, dho:]

    in_specs = [
        pl.BlockSpec((2, BN, dhi), lambda i: (0, i, 0)),
        pl.BlockSpec((din, dout), lambda i: (0, 0)),
        pl.BlockSpec((1, dout), lambda i: (0, 0)),
    ]
    args = [h2, W, b2]
    if have_si:
        in_specs.append(pl.BlockSpec((BN, 1), lambda i: (i, 0)))
        args.append(scale_in.reshape(n, 1))
    if have_pb:
        in_specs.append(pl.BlockSpec((1, din), lambda i: (0, 0)))
        args.append(pre_bias.reshape(1, din))
    if have_so:
        in_specs.append(pl.BlockSpec((BN, 1), lambda i: (i, 0)))
        args.append(scale_out.reshape(n, 1))
    return pl.pallas_call(
        body,
        grid=(n // BN,),
        in_specs=in_specs,
        out_specs=pl.BlockSpec((2, BN, dho), lambda i: (0, i, 0)),
        out_shape=jax.ShapeDtypeStruct((2, n, dho), jnp.float32),
    )(*args)


BM = 400


def _scale_split(x, dis):
    n, d = x.shape
    dh = d // 2

    def body(x_ref, s_ref, o_ref):
        xs = x_ref[...] * s_ref[...]
        o_ref[0] = xs[:, :dh]
        o_ref[1] = xs[:, dh:]

    return pl.pallas_call(
        body,
        grid=(n // BM,),
        in_specs=[
            pl.BlockSpec((BM, d), lambda i: (i, 0)),
            pl.BlockSpec((BM, 1), lambda i: (i, 0)),
        ],
        out_specs=pl.BlockSpec((2, BM, dh), lambda i: (0, i, 0)),
        out_shape=jax.ShapeDtypeStruct((2, NP, dh), jnp.float32),
    )(x, dis.reshape(dis.shape[0], 1))


def _eltwise_split(v2, dis, b, *, relu, dis_out):
    _, n, dh = v2.shape

    def body(v_ref, s_ref, b_ref, o_ref):
        s = s_ref[...]
        for c in range(2):
            y = v_ref[c] * s + b_ref[c]
            if relu:
                y = jnp.maximum(y, 0.0)
            if dis_out:
                y = y * s
            o_ref[c] = y

    return pl.pallas_call(
        body,
        grid=(n // BN,),
        in_specs=[
            pl.BlockSpec((2, BN, dh), lambda i: (0, i, 0)),
            pl.BlockSpec((BN, 1), lambda i: (i, 0)),
            pl.BlockSpec((2, 1, dh), lambda i: (0, 0, 0)),
        ],
        out_specs=pl.BlockSpec((2, BN, dh), lambda i: (0, i, 0)),
        out_shape=jax.ShapeDtypeStruct((2, n, dh), jnp.float32),
    )(v2, dis.reshape(n, 1), b.reshape(2, 1, dh))


def _merge_final(v2, dis, b):
    _, n, dh = v2.shape

    def body(v_ref, s_ref, b_ref, o_ref):
        y = jnp.concatenate([v_ref[0], v_ref[1]], axis=1)
        o_ref[...] = y * s_ref[...] + b_ref[...]

    return pl.pallas_call(
        body,
        grid=(N_NODES // BM,),
        in_specs=[
            pl.BlockSpec((2, BM, dh), lambda i: (0, i, 0)),
            pl.BlockSpec((BM, 1), lambda i: (i, 0)),
            pl.BlockSpec((1, 2 * dh), lambda i: (0, 0)),
        ],
        out_specs=pl.BlockSpec((BM, 2 * dh), lambda i: (i, 0)),
        out_shape=jax.ShapeDtypeStruct((N_NODES, 2 * dh), jnp.float32),
    )(v2, dis.reshape(n, 1), b.reshape(1, 2 * dh))


DW = 16


def _sc_degree(dstr):
    mesh = plsc.VectorSubcoreMesh(core_axis_name="c", subcore_axis_name="s")
    nchd = NCH // 2
    nblkd = nchd // KB

    @functools.partial(
        pl.kernel,
        out_type=jax.ShapeDtypeStruct((2, NP, DW), jnp.float32),
        mesh=mesh,
        compiler_params=pltpu.CompilerParams(use_tc_tiling_on_sc=False),
        scratch_types=[
            pltpu.MemorySpace.VMEM_SHARED((NP, DW), jnp.float32),
            pltpu.MemorySpace.VMEM((2, KB, EC), jnp.int32),
            pltpu.MemorySpace.VMEM((EC, DW), jnp.float32),
            pltpu.MemorySpace.VMEM((RPT, DW), jnp.float32),
            pltpu.SemaphoreType.DMA,
            pltpu.SemaphoreType.DMA,
            pltpu.SemaphoreType.DMA,
        ],
    )
    def deg(dst_hbm, out_hbm, acc, didx, ones_v, zero_v, isem0, isem1, ssem):
        cc = lax.axis_index("c")
        sid = lax.axis_index("s")
        row0 = sid * RPT
        isems = (isem0, isem1)

        def idx_prefetch(p, bb):
            pltpu.async_copy(
                dst_hbm.at[sid, pl.ds(cc * nchd + bb * KB, KB)],
                didx.at[p], isems[p])

        def idx_drain(p, bb):
            pltpu.make_async_copy(
                dst_hbm.at[sid, pl.ds(cc * nchd + bb * KB, KB)],
                didx.at[p], isems[p]).wait()

        idx_prefetch(0, 0)

        def fill_ones(r, carry):
            ones_v[r, :] = jnp.ones((DW,), jnp.float32)
            return carry

        def fill_zero(r, carry):
            zero_v[r, :] = jnp.zeros((DW,), jnp.float32)
            return carry

        lax.fori_loop(0, EC, fill_ones, 0)
        lax.fori_loop(0, RPT, fill_zero, 0)
        pltpu.sync_copy(zero_v, acc.at[pl.ds(row0, RPT)])
        plsc.subcore_barrier()

        def blockpair(go, carry):
            for p in (0, 1):
                bb = 2 * go + p
                idx_drain(p, bb)
                if p == 0:
                    idx_prefetch(1, bb + 1)
                else:
                    @pl.when(go < nblkd // 2 - 1)
                    def _():
                        idx_prefetch(0, bb + 1)
                for j in range(KB):
                    pltpu.async_copy(ones_v, acc.at[didx.at[p, j]], ssem,
                                     add=True)
                for j in range(KB):
                    pltpu.make_async_copy(ones_v, acc.at[didx.at[p, 0]],
                                          ssem).wait()
            return carry

        lax.fori_loop(0, nblkd // 2, blockpair, 0)
        plsc.subcore_barrier()
        pltpu.sync_copy(acc.at[pl.ds(row0, RPT)],
                        out_hbm.at[cc, pl.ds(row0, RPT)])

    return deg(dstr)


def _sc_agg(u2, srcx, dstr, *, dh):
    u2f = u2.reshape(2 * NP, dh)
    mesh = plsc.VectorSubcoreMesh(core_axis_name="c", subcore_axis_name="s")
    ring = 2 if dh == 128 else 5

    sems = [pltpu.SemaphoreType.DMA] * (2 + 2 * ring)

    @functools.partial(
        pl.kernel,
        out_type=jax.ShapeDtypeStruct((2, NP, dh), jnp.float32),
        mesh=mesh,
        compiler_params=pltpu.CompilerParams(use_tc_tiling_on_sc=False),
        scratch_types=[
            pltpu.MemorySpace.VMEM_SHARED((NP, dh), jnp.float32),
            pltpu.MemorySpace.VMEM((2, KB, EC), jnp.int32),
            pltpu.MemorySpace.VMEM((2, KB, EC), jnp.int32),
            pltpu.MemorySpace.VMEM((ring, EC, dh), jnp.float32),
        ] + sems,
    )
    def agg(*refs):
        (u_hbm, srcx_hbm, dst_hbm, out_hbm, acc, sidx, didx, rows) = refs[:8]
        isems = refs[8:10]
        gsems = refs[10:10 + ring]
        ssems = refs[10 + ring:10 + 2 * ring]
        cc = lax.axis_index("c")
        sid = lax.axis_index("s")
        row0 = sid * RPT

        def idx_prefetch(p, bb):
            pltpu.async_copy(srcx_hbm.at[cc, sid, pl.ds(bb * KB, KB)],
                             sidx.at[p], isems[p])
            pltpu.async_copy(dst_hbm.at[sid, pl.ds(bb * KB, KB)],
                             didx.at[p], isems[p])

        def idx_drain(p, bb):
            pltpu.make_async_copy(srcx_hbm.at[cc, sid, pl.ds(bb * KB, KB)],
                                  sidx.at[p], isems[p]).wait()
            pltpu.make_async_copy(dst_hbm.at[sid, pl.ds(bb * KB, KB)],
                                  didx.at[p], isems[p]).wait()

        def gather_start(p, j):
            q = j % ring
            pltpu.async_copy(u_hbm.at[sidx.at[p, j]], rows.at[q], gsems[q])

        def gather_wait(p, j):
            q = j % ring
            pltpu.make_async_copy(u_hbm.at[sidx.at[p, j]], rows.at[q],
                                  gsems[q]).wait()

        def scatter_start(p, j):
            q = j % ring
            pltpu.async_copy(rows.at[q], acc.at[didx.at[p, j]], ssems[q],
                             add=True)

        def scatter_wait(q):
            pltpu.make_async_copy(rows.at[q], acc.at[didx.at[0, 0]],
                                  ssems[q]).wait()

        idx_prefetch(0, 0)

        pltpu.sync_copy(u_hbm.at[pl.ds(cc * NP + row0, RPT)],
                        acc.at[pl.ds(row0, RPT)])
        plsc.subcore_barrier()

        G = 1 if ring == 2 else 3

        def blockpair(go, carry):
            for p in (0, 1):
                bb = 2 * go + p
                idx_drain(p, bb)
                for j in range(KB):
                    q = j % ring
                    if p == 0 and j < ring:
                        @pl.when(go > 0)
                        def _():
                            scatter_wait(q)
                    else:
                        scatter_wait(q)
                    gather_start(p, j)
                    if j >= G:
                        gather_wait(p, j - G)
                        scatter_start(p, j - G)
                    elif p == 1:
                        gather_wait(0, KB - G + j)
                        scatter_start(0, KB - G + j)
                    else:
                        @pl.when(go > 0)
                        def _():
                            gather_wait(1, KB - G + j)
                            scatter_start(1, KB - G + j)
                    if j == ring - 1:
                        if p == 0:
                            idx_prefetch(1, bb + 1)
                        else:
                            @pl.when(go < NBLK // 2 - 1)
                            def _():
                                idx_prefetch(0, bb + 1)
            return carry

        lax.fori_loop(0, NBLK // 2, blockpair, 0)
        for t in range(G):
            gather_wait(1, KB - G + t)
            scatter_start(1, KB - G + t)
        for q in range(ring):
            scatter_wait(q)
        plsc.subcore_barrier()

        pltpu.sync_copy(acc.at[pl.ds(row0, RPT)],
                        out_hbm.at[cc, pl.ds(row0, RPT)])

    return agg(u2f, srcx, dstr)


def kernel(x, edge_index, W1, b1, W2, b2, W3, b3, W4, b4, W5, b5,
           W6, b6, W7, b7, W8, b8, W9, b9, W10, b10):
    src = edge_index[0].astype(jnp.int32)
    dst = edge_index[1].astype(jnp.int32)

    srcx = jnp.stack([src, src + NP]).reshape(2, NS, NCH, EC)
    dstr = dst.reshape(NS, NCH, EC)

    cnt = _sc_degree(dstr)
    deg_pad = cnt[0, :, 0] + cnt[1, :, 0] + 1.0
    dis_pad = deg_pad ** -0.5

    u1 = _scale_split(x, dis_pad)
    v1 = _sc_agg(u1, srcx, dstr, dh=64)
    h1 = _dense_split(v1, W1, b1, relu=True, scale_in=dis_pad)

    t2 = _dense_split(h1, W2, jnp.zeros_like(b2), relu=False,
                      scale_out=dis_pad)
    v2 = _sc_agg(t2, srcx, dstr, dh=128)
    t3 = _dense_split(v2, W3, jnp.zeros_like(b3), relu=False,
                      scale_in=dis_pad, pre_bias=b2, pre_relu=True,
                      scale_out=dis_pad)
    v3 = _sc_agg(t3, srcx, dstr, dh=64)
    t4 = _dense_split(v3, W4, jnp.zeros_like(b4), relu=False,
                      scale_in=dis_pad, pre_bias=b3, pre_relu=True,
                      scale_out=dis_pad)
    v4 = _sc_agg(t4, srcx, dstr, dh=32)
    t5 = _dense_split(v4, W5, jnp.zeros_like(b5), relu=False,
                      scale_in=dis_pad, pre_bias=b4, pre_relu=True,
                      scale_out=dis_pad)
    v5 = _sc_agg(t5, srcx, dstr, dh=16)
    u6 = _eltwise_split(v5, dis_pad, b5, relu=True, dis_out=True)
    v6 = _sc_agg(u6, srcx, dstr, dh=16)
    u7 = _dense_split(v6, W6, b6, relu=True, scale_in=dis_pad,
                      scale_out=dis_pad)
    v7 = _sc_agg(u7, srcx, dstr, dh=32)
    u8 = _dense_split(v7, W7, b7, relu=True, scale_in=dis_pad,
                      scale_out=dis_pad)
    v8 = _sc_agg(u8, srcx, dstr, dh=64)
    u9 = _dense_split(v8, W8, b8, relu=True, scale_in=dis_pad,
                      scale_out=dis_pad)
    v9 = _sc_agg(u9, srcx, dstr, dh=128)
    h9 = _dense_split(v9, W9, b9, relu=True, scale_in=dis_pad)
    t10 = _dense_split(h9, W10, jnp.zeros_like(b10), relu=False,
                       scale_out=dis_pad)
    v10 = _sc_agg(t10, srcx, dstr, dh=64)
    return _merge_final(v10, dis_pad, b10)
